# baseline (speedup 1.0000x reference)
import functools

import jax
import jax.numpy as jnp
from jax import lax
from jax.experimental import pallas as pl
from jax.experimental.pallas import tpu as pltpu
from jax.experimental.pallas import tpu_sc as plsc

NUM_EMB = 1_000_000
DIM = 64
PAD_DIM = 128
BATCH = 16384
FIELDS = 26

NC = 2
NS = 16
NW = NC * NS
B_PER_W = BATCH // NW
CHUNK_B = 8
NCHUNK = B_PER_W // CHUNK_B

_mesh = plsc.VectorSubcoreMesh(core_axis_name="c", subcore_axis_name="s")


@functools.partial(
    pl.kernel,
    out_type=jax.ShapeDtypeStruct((BATCH, FIELDS, DIM), jnp.float32),
    mesh=_mesh,
    scratch_types=[
        pltpu.VMEM((CHUNK_B, FIELDS), jnp.int32),
        pltpu.VMEM((CHUNK_B, FIELDS, PAD_DIM), jnp.float32),
        pltpu.VMEM((CHUNK_B, FIELDS, DIM), jnp.float32),
        pltpu.SemaphoreType.DMA,
    ],
)
def _gather_kernel(idx_hbm, table_hbm, out_hbm, idx_v, rows_v, out_v, sem):
    wid = lax.axis_index("s") * NC + lax.axis_index("c")
    base = wid * B_PER_W

    def body(c, carry):
        s = base + c * CHUNK_B
        pltpu.sync_copy(idx_hbm.at[pl.ds(s, CHUNK_B), :], idx_v)
        copies = [
            pltpu.async_copy(table_hbm.at[idx_v.at[i, :]], rows_v.at[i], sem)
            for i in range(CHUNK_B)
        ]
        for cp in copies:
            cp.wait()
        for i in range(CHUNK_B):
            for f in range(FIELDS):
                for k in range(DIM // 16):
                    sl = pl.ds(k * 16, 16)
                    out_v[i, f, sl] = rows_v[i, f, sl]
        pltpu.sync_copy(out_v, out_hbm.at[pl.ds(s, CHUNK_B)])
        return carry

    lax.fori_loop(0, NCHUNK, body, 0)


def kernel(token_ids, weights):
    wpad = jnp.pad(weights, ((0, 0), (0, PAD_DIM - DIM)))
    return _gather_kernel(token_ids.astype(jnp.int32), wpad)
